# Initial kernel scaffold; baseline (speedup 1.0000x reference)
import jax
import jax.numpy as jnp
from jax.experimental import pallas as pl


def kernel(cls_out_0, cls_out_1, cls_out_2, cls_out_3, cls_out_4, box_out_0, box_out_1, box_out_2, box_out_3, box_out_4, anchor_boxes, img_scale):
    raise NotImplementedError("write your pallas kernel here")


# speedup vs baseline: 1.2365x; 1.2365x over previous
import jax
import jax.numpy as jnp
from jax import lax
from jax.experimental import pallas as pl
from jax.experimental.pallas import tpu as pltpu

_NUM_CLASSES = 20
_NUM_GMM = 3
_MAX_DET_POINTS = 5000
_MAX_DET_PER_IMAGE = 100
_IOU_THRESH = 0.5
_MASK_VAL = -1e4
_NEG = -1e30

_ROWS = 40
_LANES = 128
_NPAD = _ROWS * _LANES


def _nms_kernel(in_ref, out_ref, scr):
    ty = in_ref[0, 0]
    tx = in_ref[0, 1]
    th = in_ref[0, 2]
    tw = in_ref[0, 3]
    ay1 = in_ref[0, 4]
    ax1 = in_ref[0, 5]
    ay2 = in_ref[0, 6]
    ax2 = in_ref[0, 7]
    logit = in_ref[0, 8]
    cls = in_ref[0, 9]
    img_scale = in_ref[0, 10, 0, 0]

    r_iota = lax.broadcasted_iota(jnp.int32, (_ROWS, _LANES), 0)
    c_iota = lax.broadcasted_iota(jnp.int32, (_ROWS, _LANES), 1)
    fiota = r_iota * _LANES + c_iota
    valid = fiota < _MAX_DET_POINTS

    ha = ay2 - ay1
    wa = ax2 - ax1
    yca = (ay1 + ay2) / 2.0
    xca = (ax1 + ax2) / 2.0
    w = jnp.exp(tw) * wa
    h = jnp.exp(th) * ha
    yc = ty * ha + yca
    xc = tx * wa + xca
    x1 = jnp.where(valid, xc - w / 2.0, _NEG)
    y1 = jnp.where(valid, yc - h / 2.0, _NEG)
    x2 = jnp.where(valid, xc + w / 2.0, _NEG)
    y2 = jnp.where(valid, yc + h / 2.0, _NEG)

    s0 = jnp.where(valid, jax.nn.sigmoid(logit), _NEG)

    max_coord = jnp.maximum(jnp.maximum(jnp.max(x1), jnp.max(y1)),
                            jnp.maximum(jnp.max(x2), jnp.max(y2)))
    off = cls * (max_coord + 1.0)
    nx1 = x1 + off
    ny1 = y1 + off
    nx2 = x2 + off
    ny2 = y2 + off
    area = (nx2 - nx1) * (ny2 - ny1)

    scr[0] = s0
    scr[1] = cls
    scr[2] = x1
    scr[3] = y1
    scr[4] = x2
    scr[5] = y2
    scr[6] = nx1
    scr[7] = ny1
    scr[8] = nx2
    scr[9] = ny2
    scr[10] = area

    li = lax.broadcasted_iota(jnp.int32, (1, _LANES), 1)

    def body(it, carry):
        s = scr[0]
        m = jnp.max(s)
        idx = jnp.where(s == m, fiota, jnp.int32(2 ** 30))
        i = jnp.min(idx)
        maskf = (fiota == i).astype(jnp.float32)

        score_i = jnp.sum(s * maskf)
        cls_i = jnp.sum(scr[1] * maskf)
        bx1 = jnp.sum(scr[2] * maskf)
        by1 = jnp.sum(scr[3] * maskf)
        bx2 = jnp.sum(scr[4] * maskf)
        by2 = jnp.sum(scr[5] * maskf)
        sx1 = jnp.sum(scr[6] * maskf)
        sy1 = jnp.sum(scr[7] * maskf)
        sx2 = jnp.sum(scr[8] * maskf)
        sy2 = jnp.sum(scr[9] * maskf)
        ar_i = jnp.sum(scr[10] * maskf)

        ltx = jnp.maximum(sx1, scr[6])
        lty = jnp.maximum(sy1, scr[7])
        rbx = jnp.minimum(sx2, scr[8])
        rby = jnp.minimum(sy2, scr[9])
        iw = jnp.maximum(rbx - ltx, 0.0)
        ih = jnp.maximum(rby - lty, 0.0)
        inter = iw * ih
        iou = inter / (ar_i + scr[10] - inter + 1e-8)
        scr[0] = jnp.where(iou > _IOU_THRESH, _MASK_VAL, s)

        ok = score_i > (_MASK_VAL / 2.0)
        vals = (bx1 * img_scale, by1 * img_scale, bx2 * img_scale,
                by2 * img_scale, score_i, cls_i + 1.0)
        row = jnp.zeros((1, _LANES), jnp.float32)
        for k, v in enumerate(vals):
            row = jnp.where(li == k, v, row)
        row = jnp.where(ok, row, jnp.zeros((1, _LANES), jnp.float32))
        out_ref[0, pl.ds(it, 1), :] = row
        return carry

    lax.fori_loop(0, _MAX_DET_PER_IMAGE, body, 0)


def _mix_gmm(out, num_gmm):
    o = jnp.transpose(out, (0, 2, 3, 1))
    mean, _, weights = jnp.split(o, 3, axis=-1)
    b, hh, ww, ck = weights.shape
    c = ck // num_gmm
    wsm = jax.nn.softmax(weights.reshape(b, hh, ww, c, num_gmm), axis=-1)
    return (wsm.reshape(b, hh, ww, ck) * mean).reshape(b, hh, ww, c, num_gmm).sum(-1)


def _pad_plane(x):
    b = x.shape[0]
    x = jnp.pad(x, ((0, 0), (0, _NPAD - _MAX_DET_POINTS)))
    return x.reshape(b, _ROWS, _LANES)


def kernel(cls_out_0, cls_out_1, cls_out_2, cls_out_3, cls_out_4,
           box_out_0, box_out_1, box_out_2, box_out_3, box_out_4,
           anchor_boxes, img_scale):
    cls_outs = [cls_out_0, cls_out_1, cls_out_2, cls_out_3, cls_out_4]
    box_outs = [box_out_0, box_out_1, box_out_2, box_out_3, box_out_4]
    batch = cls_outs[0].shape[0]
    cls_all = jnp.concatenate(
        [_mix_gmm(o, _NUM_GMM).reshape(batch, -1, _NUM_CLASSES) for o in cls_outs], axis=1)
    box_all = jnp.concatenate(
        [_mix_gmm(o, _NUM_GMM).reshape(batch, -1, 4) for o in box_outs], axis=1)

    logits, topk_idx = lax.top_k(cls_all.reshape(batch, -1), _MAX_DET_POINTS)
    indices_all = topk_idx // _NUM_CLASSES
    classes_all = topk_idx % _NUM_CLASSES
    box_sel = jnp.take_along_axis(box_all, indices_all[:, :, None], axis=1)
    anch_sel = anchor_boxes[indices_all]

    planes = [
        box_sel[..., 0], box_sel[..., 1], box_sel[..., 2], box_sel[..., 3],
        anch_sel[..., 0], anch_sel[..., 1], anch_sel[..., 2], anch_sel[..., 3],
        logits, classes_all.astype(jnp.float32),
    ]
    stacked = jnp.stack([_pad_plane(p) for p in planes], axis=1)
    scale_plane = jnp.broadcast_to(
        img_scale[:, None, None, None], (batch, 1, _ROWS, _LANES))
    stacked = jnp.concatenate([stacked, scale_plane], axis=1)

    out = pl.pallas_call(
        _nms_kernel,
        grid=(batch,),
        in_specs=[pl.BlockSpec((1, 11, _ROWS, _LANES), lambda b: (b, 0, 0, 0))],
        out_specs=pl.BlockSpec((1, _MAX_DET_PER_IMAGE, _LANES), lambda b: (b, 0, 0)),
        out_shape=jax.ShapeDtypeStruct((batch, _MAX_DET_PER_IMAGE, _LANES), jnp.float32),
        scratch_shapes=[pltpu.VMEM((11, _ROWS, _LANES), jnp.float32)],
    )(stacked)
    return out[:, :, :6]
